# baseline (speedup 1.0000x reference)
import functools

import jax
import jax.numpy as jnp
from jax import lax
from jax.experimental import pallas as pl
from jax.experimental.pallas import tpu as pltpu
from jax.experimental.pallas import tpu_sc as plsc

N = 10000
E = 160000
D = 256
C = 40
H = 128
NC = 2
NS = 16
LN = 16

HQ = 128
NQ = D // HQ

ACC_ROWS = 10240
EPT = E // NS
CH = 128
NCHUNK = EPT // CH
TAIL = EPT - NCHUNK * CH
ZPT = ACC_ROWS // NS

@functools.cache
def _mesh():
    return plsc.VectorSubcoreMesh(core_axis_name="c", subcore_axis_name="s",
                                  num_cores=NC, num_subcores=NS)


DCH = 2000
RED = ACC_ROWS // NS


def _deg_body(dst_hbm, out_hbm, dbuf, hist, tmp, accb, staging):
    c = lax.axis_index("c")
    s = lax.axis_index("s")
    ones = jnp.ones((LN,), jnp.float32)
    zeros = jnp.zeros((LN,), jnp.float32)

    def _zh(i, _):
        hist[pl.ds(i * LN, LN)] = zeros
        return 0

    lax.fori_loop(0, ACC_ROWS // LN, _zh, 0)

    def _stage(k, _):
        pltpu.sync_copy(dst_hbm.at[pl.ds(s * EPT + k * DCH, DCH)], dbuf)

        def _hist16(j, _):
            plsc.addupdate_scatter(hist, [dbuf[pl.ds(j * LN, LN)]], ones)
            return 0

        lax.fori_loop(0, DCH // LN, _hist16, 0)
        return 0

    lax.fori_loop(0, EPT // DCH, _stage, 0)
    pltpu.sync_copy(hist, staging.at[pl.ds(s * ACC_ROWS, ACC_ROWS)])
    plsc.subcore_barrier()

    @pl.when(c == 0)
    def _():
        def _zb(i, _):
            accb[pl.ds(i * LN, LN)] = zeros
            return 0

        lax.fori_loop(0, RED // LN, _zb, 0)

        def _red(s2, _):
            pltpu.sync_copy(
                staging.at[pl.ds(s2 * ACC_ROWS + s * RED, RED)], tmp)

            def _add16(i, _):
                accb[pl.ds(i * LN, LN)] = accb[pl.ds(i * LN, LN)] \
                    + tmp[pl.ds(i * LN, LN)]
                return 0

            lax.fori_loop(0, RED // LN, _add16, 0)
            return 0

        lax.fori_loop(0, NS, _red, 0)
        pltpu.sync_copy(accb, out_hbm.at[pl.ds(s * RED, RED)])


@functools.cache
def _deg_kernel():
    return pl.kernel(
        _deg_body,
        out_type=jax.ShapeDtypeStruct((ACC_ROWS,), jnp.float32),
        mesh=_mesh(),
        compiler_params=pltpu.CompilerParams(needs_layout_passes=False),
        scratch_types=[
            pltpu.VMEM((DCH,), jnp.int32),
            pltpu.VMEM((ACC_ROWS,), jnp.float32),
            pltpu.VMEM((RED,), jnp.float32),
            pltpu.VMEM((RED,), jnp.float32),
            pltpu.VMEM_SHARED((NS * ACC_ROWS,), jnp.float32),
        ],
    )


def _deg_call(dst):
    return _deg_kernel()(dst)


def _agg_body(y_hbm, src_hbm, dst_hbm, out_hbm,
              rows0, rows1, rows_t, sbuf0, sbuf1, dbuf0, dbuf1,
              sbuf_t, dbuf_t, acc,
              gsem0, gsem1, ssem0, ssem1, tsem):
    c = lax.axis_index("c")
    s = lax.axis_index("s")
    off = c * N
    rows = (rows0, rows1)
    sbuf = (sbuf0, sbuf1)
    dbuf = (dbuf0, dbuf1)
    gsem = (gsem0, gsem1)
    ssem = (ssem0, ssem1)

    def _zrow(i, _):
        def _z16(j, _):
            rows0[i, pl.ds(j * LN, LN)] = jnp.zeros((LN,), jnp.float32)
            return 0

        lax.fori_loop(0, H // LN, _z16, 0)
        return 0

    lax.fori_loop(0, CH, _zrow, 0)

    def _zero(j, _):
        pltpu.sync_copy(rows0, acc.at[pl.ds(s * ZPT + j * CH, CH)])
        return 0

    lax.fori_loop(0, ZPT // CH, _zero, 0)
    plsc.subcore_barrier()

    def _stage(k, p):
        e0 = s * EPT + k * CH
        pltpu.sync_copy(src_hbm.at[pl.ds(e0, CH)], sbuf[p])
        pltpu.sync_copy(dst_hbm.at[pl.ds(e0, CH)], dbuf[p])

        def _add_off(i, _):
            sbuf[p][pl.ds(i * LN, LN)] = sbuf[p][pl.ds(i * LN, LN)] + off
            return 0

        lax.fori_loop(0, CH // LN, _add_off, 0)
        pltpu.async_copy(y_hbm.at[sbuf[p]], rows[p], gsem[p])

    _stage(0, 0)

    def _pipe(kk, _):
        for p in (0, 1):
            q = 1 - p
            k = 2 * kk + p
            pltpu.make_async_copy(y_hbm.at[sbuf[p]], rows[p], gsem[p]).wait()

            @pl.when(k > 0)
            def _():
                pltpu.make_async_copy(
                    rows[q], acc.at[dbuf[q]], ssem[q]).wait()

            pltpu.async_copy(rows[p], acc.at[dbuf[p]], ssem[p], add=True)

            @pl.when(k < NCHUNK - 1)
            def _():
                _stage(k + 1, q)
        return 0

    lax.fori_loop(0, NCHUNK // 2, _pipe, 0)
    pltpu.make_async_copy(rows1, acc.at[dbuf1], ssem1).wait()

    e0 = s * EPT + NCHUNK * CH
    pltpu.sync_copy(src_hbm.at[pl.ds(e0, TAIL)], sbuf_t)
    pltpu.sync_copy(dst_hbm.at[pl.ds(e0, TAIL)], dbuf_t)

    def _add_off_t(i, _):
        sbuf_t[pl.ds(i * LN, LN)] = sbuf_t[pl.ds(i * LN, LN)] + off
        return 0

    lax.fori_loop(0, TAIL // LN, _add_off_t, 0)
    pltpu.async_copy(y_hbm.at[sbuf_t], rows_t, tsem).wait()
    pltpu.sync_copy(rows_t, acc.at[dbuf_t], add=True)

    plsc.subcore_barrier()
    pltpu.sync_copy(acc.at[pl.ds(s * ZPT, ZPT)],
                    out_hbm.at[pl.ds(c * ACC_ROWS + s * ZPT, ZPT)])


@functools.cache
def _agg_kernel():
    return pl.kernel(
        _agg_body,
        out_type=jax.ShapeDtypeStruct((NC * ACC_ROWS, H), jnp.float32),
        mesh=_mesh(),
        scratch_types=[
            pltpu.VMEM((CH, H), jnp.float32),
            pltpu.VMEM((CH, H), jnp.float32),
            pltpu.VMEM((TAIL, H), jnp.float32),
            pltpu.VMEM((CH,), jnp.int32),
            pltpu.VMEM((CH,), jnp.int32),
            pltpu.VMEM((CH,), jnp.int32),
            pltpu.VMEM((CH,), jnp.int32),
            pltpu.VMEM((TAIL,), jnp.int32),
            pltpu.VMEM((TAIL,), jnp.int32),
            pltpu.VMEM_SHARED((ACC_ROWS, H), jnp.float32),
            pltpu.SemaphoreType.DMA,
            pltpu.SemaphoreType.DMA,
            pltpu.SemaphoreType.DMA,
            pltpu.SemaphoreType.DMA,
            pltpu.SemaphoreType.DMA,
        ],
    )


def _agg_call(y, src2d, dst2d):
    return _agg_kernel()(y, src2d, dst2d)


MB = 1024
NBLK = (N + MB - 1) // MB


def _tc_a1_body(x_ref, w_ref, o_ref):
    o_ref[...] = jnp.dot(x_ref[...], w_ref[...],
                         preferred_element_type=jnp.float32)


def _tc_a1(x, w1):
    return pl.pallas_call(
        _tc_a1_body,
        grid=(NBLK,),
        in_specs=[
            pl.BlockSpec((MB, D), lambda i: (i, 0)),
            pl.BlockSpec((D, D), lambda i: (0, 0)),
        ],
        out_specs=pl.BlockSpec((MB, D), lambda i: (i, 0)),
        out_shape=jax.ShapeDtypeStruct((N, D), jnp.float32),
    )(x, w1)


def _tc_a2_body(xw_ref, deg_ref, o_ref, dinv_ref):
    deg = deg_ref[...]
    dinv = jnp.where(deg > 0, lax.rsqrt(deg), 0.0)
    y = xw_ref[...] * dinv
    for q in range(NQ):
        o_ref[q] = y[:, q * HQ:(q + 1) * HQ]
    dinv_ref[...] = dinv


def _tc_a2(xw, deg2):
    return pl.pallas_call(
        _tc_a2_body,
        grid=(NBLK,),
        in_specs=[
            pl.BlockSpec((MB, D), lambda i: (i, 0)),
            pl.BlockSpec((MB, 1), lambda i: (i, 0)),
        ],
        out_specs=[
            pl.BlockSpec((NQ, MB, HQ), lambda i: (0, i, 0)),
            pl.BlockSpec((MB, 1), lambda i: (i, 0)),
        ],
        out_shape=[
            jax.ShapeDtypeStruct((NQ, N, HQ), jnp.float32),
            jax.ShapeDtypeStruct((N, 1), jnp.float32),
        ],
    )(xw, deg2)


def _tc_b_body(agg_ref, dinv_ref, b1_ref, a1_ref, w2_ref, rw_ref, rb_ref,
               y2_ref, r_ref):
    dinv = dinv_ref[...]
    pre = jnp.concatenate([agg_ref[q] for q in range(NQ)], axis=1) * dinv \
        + b1_ref[...]
    h = jnp.where(pre >= 0, pre, a1_ref[...] * pre)
    y2 = jnp.dot(h, w2_ref[...], preferred_element_type=jnp.float32) * dinv
    for q in range(NQ):
        y2_ref[q] = y2[:, q * HQ:(q + 1) * HQ]
    r_ref[...] = jnp.dot(h, rw_ref[...], preferred_element_type=jnp.float32) \
        + rb_ref[...]


def _tc_b(agg1, dinv2, b1, a1, w2, rw, rb):
    return pl.pallas_call(
        _tc_b_body,
        grid=(NBLK,),
        in_specs=[
            pl.BlockSpec((NQ, MB, HQ), lambda i: (0, i, 0)),
            pl.BlockSpec((MB, 1), lambda i: (i, 0)),
            pl.BlockSpec((1, D), lambda i: (0, 0)),
            pl.BlockSpec((1, D), lambda i: (0, 0)),
            pl.BlockSpec((D, D), lambda i: (0, 0)),
            pl.BlockSpec((D, D), lambda i: (0, 0)),
            pl.BlockSpec((1, D), lambda i: (0, 0)),
        ],
        out_specs=[
            pl.BlockSpec((NQ, MB, HQ), lambda i: (0, i, 0)),
            pl.BlockSpec((MB, D), lambda i: (i, 0)),
        ],
        out_shape=[
            jax.ShapeDtypeStruct((NQ, N, HQ), jnp.float32),
            jax.ShapeDtypeStruct((N, D), jnp.float32),
        ],
    )(agg1, dinv2, b1, a1, w2, rw, rb)


def _tc_c_body(agg_ref, dinv_ref, b2_ref, a2_ref, r_ref, wc_ref, bc_ref, o_ref):
    dinv = dinv_ref[...]
    pre = jnp.concatenate([agg_ref[q] for q in range(NQ)], axis=1) * dinv \
        + b2_ref[...] + r_ref[...]
    h2 = jnp.where(pre >= 0, pre, a2_ref[...] * pre)
    o_ref[...] = jnp.dot(h2, wc_ref[...], preferred_element_type=jnp.float32) \
        + bc_ref[...]


def _tc_c(agg2, dinv2, b2, a2, r, wc, bc):
    return pl.pallas_call(
        _tc_c_body,
        grid=(NBLK,),
        in_specs=[
            pl.BlockSpec((NQ, MB, HQ), lambda i: (0, i, 0)),
            pl.BlockSpec((MB, 1), lambda i: (i, 0)),
            pl.BlockSpec((1, D), lambda i: (0, 0)),
            pl.BlockSpec((1, D), lambda i: (0, 0)),
            pl.BlockSpec((MB, D), lambda i: (i, 0)),
            pl.BlockSpec((D, C), lambda i: (0, 0)),
            pl.BlockSpec((1, C), lambda i: (0, 0)),
        ],
        out_specs=pl.BlockSpec((MB, C), lambda i: (i, 0)),
        out_shape=jax.ShapeDtypeStruct((N, C), jnp.float32),
    )(agg2, dinv2, b2, a2, r, wc, bc)


def kernel(x, edge_index, W1, b1, a1, W2, b2, a2, resW, resb, Wc, bc):
    src = edge_index[0]
    dst = edge_index[1]
    xw = _tc_a1(x, W1)
    deg = _deg_call(dst)
    deg2 = deg[:N].reshape(N, 1)
    y1, dinv2 = _tc_a2(xw, deg2)
    agg1 = _agg_call(y1.reshape(NQ * N, HQ), src,
                     dst).reshape(NQ, ACC_ROWS, HQ)
    y2, r = _tc_b(agg1, dinv2, b1.reshape(1, D), a1.reshape(1, D),
                  W2, resW, resb.reshape(1, D))
    agg2 = _agg_call(y2.reshape(NQ * N, HQ), src,
                     dst).reshape(NQ, ACC_ROWS, HQ)
    return _tc_c(agg2, dinv2, b2.reshape(1, D), a2.reshape(1, D),
                 r, Wc, bc.reshape(1, C))


# speedup vs baseline: 1.0032x; 1.0032x over previous
import functools

import jax
import jax.numpy as jnp
from jax import lax
from jax.experimental import pallas as pl
from jax.experimental.pallas import tpu as pltpu
from jax.experimental.pallas import tpu_sc as plsc

N = 10000
E = 160000
D = 256
C = 40
H = 128
NC = 2
NS = 16
LN = 16

HQ = 128
NQ = D // HQ

ACC_ROWS = 10240
EPT = E // NS
CH = 128
NCHUNK = EPT // CH
TAIL = EPT - NCHUNK * CH
WPT = N // NS
ZPT = ACC_ROWS // NS

@functools.cache
def _mesh():
    return plsc.VectorSubcoreMesh(core_axis_name="c", subcore_axis_name="s",
                                  num_cores=NC, num_subcores=NS)


DCH = 2000
RED = ACC_ROWS // NS


def _deg_body(dst_hbm, out_hbm, dbuf, hist, tmp, accb, staging):
    c = lax.axis_index("c")
    s = lax.axis_index("s")
    ones = jnp.ones((LN,), jnp.float32)
    zeros = jnp.zeros((LN,), jnp.float32)

    def _zh(i, _):
        hist[pl.ds(i * LN, LN)] = zeros
        return 0

    lax.fori_loop(0, ACC_ROWS // LN, _zh, 0)

    def _stage(k, _):
        pltpu.sync_copy(dst_hbm.at[pl.ds(s * EPT + k * DCH, DCH)], dbuf)

        def _hist16(j, _):
            plsc.addupdate_scatter(hist, [dbuf[pl.ds(j * LN, LN)]], ones)
            return 0

        lax.fori_loop(0, DCH // LN, _hist16, 0)
        return 0

    lax.fori_loop(0, EPT // DCH, _stage, 0)
    pltpu.sync_copy(hist, staging.at[pl.ds(s * ACC_ROWS, ACC_ROWS)])
    plsc.subcore_barrier()

    @pl.when(c == 0)
    def _():
        def _zb(i, _):
            accb[pl.ds(i * LN, LN)] = zeros
            return 0

        lax.fori_loop(0, RED // LN, _zb, 0)

        def _red(s2, _):
            pltpu.sync_copy(
                staging.at[pl.ds(s2 * ACC_ROWS + s * RED, RED)], tmp)

            def _add16(i, _):
                accb[pl.ds(i * LN, LN)] = accb[pl.ds(i * LN, LN)] \
                    + tmp[pl.ds(i * LN, LN)]
                return 0

            lax.fori_loop(0, RED // LN, _add16, 0)
            return 0

        lax.fori_loop(0, NS, _red, 0)
        pltpu.sync_copy(accb, out_hbm.at[pl.ds(s * RED, RED)])


@functools.cache
def _deg_kernel():
    return pl.kernel(
        _deg_body,
        out_type=jax.ShapeDtypeStruct((ACC_ROWS,), jnp.float32),
        mesh=_mesh(),
        compiler_params=pltpu.CompilerParams(needs_layout_passes=False),
        scratch_types=[
            pltpu.VMEM((DCH,), jnp.int32),
            pltpu.VMEM((ACC_ROWS,), jnp.float32),
            pltpu.VMEM((RED,), jnp.float32),
            pltpu.VMEM((RED,), jnp.float32),
            pltpu.VMEM_SHARED((NS * ACC_ROWS,), jnp.float32),
        ],
    )


def _deg_call(dst):
    return _deg_kernel()(dst)


def _agg_body(y_hbm, src_hbm, dst_hbm, out_hbm,
              rows0, rows1, rows_t, sbuf0, sbuf1, dbuf0, dbuf1,
              sbuf_t, dbuf_t, acc,
              gsem0, gsem1, ssem0, ssem1, tsem):
    c = lax.axis_index("c")
    s = lax.axis_index("s")
    off = c * N
    rows = (rows0, rows1)
    sbuf = (sbuf0, sbuf1)
    dbuf = (dbuf0, dbuf1)
    gsem = (gsem0, gsem1)
    ssem = (ssem0, ssem1)

    def _zrow(i, _):
        def _z16(j, _):
            rows0[i, pl.ds(j * LN, LN)] = jnp.zeros((LN,), jnp.float32)
            return 0

        lax.fori_loop(0, H // LN, _z16, 0)
        return 0

    lax.fori_loop(0, CH, _zrow, 0)

    def _zero(j, _):
        pltpu.sync_copy(rows0, acc.at[pl.ds(s * ZPT + j * CH, CH)])
        return 0

    lax.fori_loop(0, ZPT // CH, _zero, 0)
    plsc.subcore_barrier()

    def _stage(k, p):
        e0 = s * EPT + k * CH
        pltpu.sync_copy(src_hbm.at[pl.ds(e0, CH)], sbuf[p])
        pltpu.sync_copy(dst_hbm.at[pl.ds(e0, CH)], dbuf[p])

        def _add_off(i, _):
            sbuf[p][pl.ds(i * LN, LN)] = sbuf[p][pl.ds(i * LN, LN)] + off
            return 0

        lax.fori_loop(0, CH // LN, _add_off, 0)
        pltpu.async_copy(y_hbm.at[sbuf[p]], rows[p], gsem[p])

    _stage(0, 0)

    def _pipe(kk, _):
        for p in (0, 1):
            q = 1 - p
            k = 2 * kk + p
            pltpu.make_async_copy(y_hbm.at[sbuf[p]], rows[p], gsem[p]).wait()

            @pl.when(k > 0)
            def _():
                pltpu.make_async_copy(
                    rows[q], acc.at[dbuf[q]], ssem[q]).wait()

            pltpu.async_copy(rows[p], acc.at[dbuf[p]], ssem[p], add=True)

            @pl.when(k < NCHUNK - 1)
            def _():
                _stage(k + 1, q)
        return 0

    lax.fori_loop(0, NCHUNK // 2, _pipe, 0)
    pltpu.make_async_copy(rows1, acc.at[dbuf1], ssem1).wait()

    e0 = s * EPT + NCHUNK * CH
    pltpu.sync_copy(src_hbm.at[pl.ds(e0, TAIL)], sbuf_t)
    pltpu.sync_copy(dst_hbm.at[pl.ds(e0, TAIL)], dbuf_t)

    def _add_off_t(i, _):
        sbuf_t[pl.ds(i * LN, LN)] = sbuf_t[pl.ds(i * LN, LN)] + off
        return 0

    lax.fori_loop(0, TAIL // LN, _add_off_t, 0)
    pltpu.async_copy(y_hbm.at[sbuf_t], rows_t, tsem).wait()
    pltpu.sync_copy(rows_t, acc.at[dbuf_t], add=True)

    plsc.subcore_barrier()
    pltpu.sync_copy(acc.at[pl.ds(s * ZPT, ZPT)],
                    out_hbm.at[pl.ds(c * ACC_ROWS + s * ZPT, ZPT)])


@functools.cache
def _agg_kernel():
    return pl.kernel(
        _agg_body,
        out_type=jax.ShapeDtypeStruct((NC * ACC_ROWS, H), jnp.float32),
        mesh=_mesh(),
        scratch_types=[
            pltpu.VMEM((CH, H), jnp.float32),
            pltpu.VMEM((CH, H), jnp.float32),
            pltpu.VMEM((TAIL, H), jnp.float32),
            pltpu.VMEM((CH,), jnp.int32),
            pltpu.VMEM((CH,), jnp.int32),
            pltpu.VMEM((CH,), jnp.int32),
            pltpu.VMEM((CH,), jnp.int32),
            pltpu.VMEM((TAIL,), jnp.int32),
            pltpu.VMEM((TAIL,), jnp.int32),
            pltpu.VMEM_SHARED((ACC_ROWS, H), jnp.float32),
            pltpu.SemaphoreType.DMA,
            pltpu.SemaphoreType.DMA,
            pltpu.SemaphoreType.DMA,
            pltpu.SemaphoreType.DMA,
            pltpu.SemaphoreType.DMA,
        ],
    )


def _agg_call(y, src2d, dst2d):
    return _agg_kernel()(y, src2d, dst2d)


MB = 1024
NBLK = (N + MB - 1) // MB


def _tc_a_body(x_ref, w_ref, deg_ref, o_ref, dinv_ref):
    deg = deg_ref[...]
    dinv = jnp.where(deg > 0, lax.rsqrt(deg), 0.0)
    y = jnp.dot(x_ref[...], w_ref[...], preferred_element_type=jnp.float32)
    y = y * dinv
    for q in range(NQ):
        o_ref[q] = y[:, q * HQ:(q + 1) * HQ]
    dinv_ref[...] = dinv


def _tc_a(x, w1, deg2):
    return pl.pallas_call(
        _tc_a_body,
        grid=(NBLK,),
        in_specs=[
            pl.BlockSpec((MB, D), lambda i: (i, 0)),
            pl.BlockSpec((D, D), lambda i: (0, 0)),
            pl.BlockSpec((MB, 1), lambda i: (i, 0)),
        ],
        out_specs=[
            pl.BlockSpec((NQ, MB, HQ), lambda i: (0, i, 0)),
            pl.BlockSpec((MB, 1), lambda i: (i, 0)),
        ],
        out_shape=[
            jax.ShapeDtypeStruct((NQ, N, HQ), jnp.float32),
            jax.ShapeDtypeStruct((N, 1), jnp.float32),
        ],
    )(x, w1, deg2)


def _tc_b_body(agg_ref, dinv_ref, b1_ref, a1_ref, w2_ref, rw_ref, rb_ref,
               y2_ref, r_ref):
    dinv = dinv_ref[...]
    pre = jnp.concatenate([agg_ref[q] for q in range(NQ)], axis=1) * dinv \
        + b1_ref[...]
    h = jnp.where(pre >= 0, pre, a1_ref[...] * pre)
    y2 = jnp.dot(h, w2_ref[...], preferred_element_type=jnp.float32) * dinv
    for q in range(NQ):
        y2_ref[q] = y2[:, q * HQ:(q + 1) * HQ]
    r_ref[...] = jnp.dot(h, rw_ref[...], preferred_element_type=jnp.float32) \
        + rb_ref[...]


def _tc_b(agg1, dinv2, b1, a1, w2, rw, rb):
    return pl.pallas_call(
        _tc_b_body,
        grid=(NBLK,),
        in_specs=[
            pl.BlockSpec((NQ, MB, HQ), lambda i: (0, i, 0)),
            pl.BlockSpec((MB, 1), lambda i: (i, 0)),
            pl.BlockSpec((1, D), lambda i: (0, 0)),
            pl.BlockSpec((1, D), lambda i: (0, 0)),
            pl.BlockSpec((D, D), lambda i: (0, 0)),
            pl.BlockSpec((D, D), lambda i: (0, 0)),
            pl.BlockSpec((1, D), lambda i: (0, 0)),
        ],
        out_specs=[
            pl.BlockSpec((NQ, MB, HQ), lambda i: (0, i, 0)),
            pl.BlockSpec((MB, D), lambda i: (i, 0)),
        ],
        out_shape=[
            jax.ShapeDtypeStruct((NQ, N, HQ), jnp.float32),
            jax.ShapeDtypeStruct((N, D), jnp.float32),
        ],
    )(agg1, dinv2, b1, a1, w2, rw, rb)


def _tc_c_body(agg_ref, dinv_ref, b2_ref, a2_ref, r_ref, wc_ref, bc_ref, o_ref):
    dinv = dinv_ref[...]
    pre = jnp.concatenate([agg_ref[q] for q in range(NQ)], axis=1) * dinv \
        + b2_ref[...] + r_ref[...]
    h2 = jnp.where(pre >= 0, pre, a2_ref[...] * pre)
    o_ref[...] = jnp.dot(h2, wc_ref[...], preferred_element_type=jnp.float32) \
        + bc_ref[...]


def _tc_c(agg2, dinv2, b2, a2, r, wc, bc):
    return pl.pallas_call(
        _tc_c_body,
        grid=(NBLK,),
        in_specs=[
            pl.BlockSpec((NQ, MB, HQ), lambda i: (0, i, 0)),
            pl.BlockSpec((MB, 1), lambda i: (i, 0)),
            pl.BlockSpec((1, D), lambda i: (0, 0)),
            pl.BlockSpec((1, D), lambda i: (0, 0)),
            pl.BlockSpec((MB, D), lambda i: (i, 0)),
            pl.BlockSpec((D, C), lambda i: (0, 0)),
            pl.BlockSpec((1, C), lambda i: (0, 0)),
        ],
        out_specs=pl.BlockSpec((MB, C), lambda i: (i, 0)),
        out_shape=jax.ShapeDtypeStruct((N, C), jnp.float32),
    )(agg2, dinv2, b2, a2, r, wc, bc)


def kernel(x, edge_index, W1, b1, a1, W2, b2, a2, resW, resb, Wc, bc):
    src = edge_index[0]
    dst = edge_index[1]
    deg = _deg_call(dst)
    deg2 = deg[:N].reshape(N, 1)
    y1, dinv2 = _tc_a(x, W1, deg2)
    agg1 = _agg_call(y1.reshape(NQ * N, HQ), src,
                     dst).reshape(NQ, ACC_ROWS, HQ)
    y2, r = _tc_b(agg1, dinv2, b1.reshape(1, D), a1.reshape(1, D),
                  W2, resW, resb.reshape(1, D))
    agg2 = _agg_call(y2.reshape(NQ * N, HQ), src,
                     dst).reshape(NQ, ACC_ROWS, HQ)
    return _tc_c(agg2, dinv2, b2.reshape(1, D), a2.reshape(1, D),
                 r, Wc, bc.reshape(1, C))


# speedup vs baseline: 1.0047x; 1.0014x over previous
import functools

import jax
import jax.numpy as jnp
from jax import lax
from jax.experimental import pallas as pl
from jax.experimental.pallas import tpu as pltpu
from jax.experimental.pallas import tpu_sc as plsc

N = 10000
E = 160000
D = 256
C = 40
H = 128
NC = 2
NS = 16
LN = 16

HQ = 128
NQ = D // HQ

ACC_ROWS = 10240
EPT = E // NS
CH = 128
NCHUNK = EPT // CH
TAIL = EPT - NCHUNK * CH
WPT = N // NS
ZPT = ACC_ROWS // NS

@functools.cache
def _mesh():
    return plsc.VectorSubcoreMesh(core_axis_name="c", subcore_axis_name="s",
                                  num_cores=NC, num_subcores=NS)


DCH = 2000
RED = ACC_ROWS // NS


def _deg_body(dst_hbm, out_hbm, dbuf, hist, tmp, accb, staging):
    c = lax.axis_index("c")
    s = lax.axis_index("s")
    ones = jnp.ones((LN,), jnp.float32)
    zeros = jnp.zeros((LN,), jnp.float32)

    def _zh(i, _):
        hist[pl.ds(i * LN, LN)] = zeros
        return 0

    lax.fori_loop(0, ACC_ROWS // LN, _zh, 0)

    def _stage(k, _):
        pltpu.sync_copy(dst_hbm.at[pl.ds(s * EPT + k * DCH, DCH)], dbuf)

        def _hist16(j, _):
            plsc.addupdate_scatter(hist, [dbuf[pl.ds(j * LN, LN)]], ones)
            return 0

        lax.fori_loop(0, DCH // LN, _hist16, 0)
        return 0

    lax.fori_loop(0, EPT // DCH, _stage, 0)
    pltpu.sync_copy(hist, staging.at[pl.ds(s * ACC_ROWS, ACC_ROWS)])
    plsc.subcore_barrier()

    @pl.when(c == 0)
    def _():
        def _zb(i, _):
            accb[pl.ds(i * LN, LN)] = zeros
            return 0

        lax.fori_loop(0, RED // LN, _zb, 0)

        def _red(s2, _):
            pltpu.sync_copy(
                staging.at[pl.ds(s2 * ACC_ROWS + s * RED, RED)], tmp)

            def _add16(i, _):
                accb[pl.ds(i * LN, LN)] = accb[pl.ds(i * LN, LN)] \
                    + tmp[pl.ds(i * LN, LN)]
                return 0

            lax.fori_loop(0, RED // LN, _add16, 0)
            return 0

        lax.fori_loop(0, NS, _red, 0)
        pltpu.sync_copy(accb, out_hbm.at[pl.ds(s * RED, RED)])


@functools.cache
def _deg_kernel():
    return pl.kernel(
        _deg_body,
        out_type=jax.ShapeDtypeStruct((ACC_ROWS,), jnp.float32),
        mesh=_mesh(),
        compiler_params=pltpu.CompilerParams(needs_layout_passes=False),
        scratch_types=[
            pltpu.VMEM((DCH,), jnp.int32),
            pltpu.VMEM((ACC_ROWS,), jnp.float32),
            pltpu.VMEM((RED,), jnp.float32),
            pltpu.VMEM((RED,), jnp.float32),
            pltpu.VMEM_SHARED((NS * ACC_ROWS,), jnp.float32),
        ],
    )


def _deg_call(dst):
    return _deg_kernel()(dst)


def _agg_body(y_hbm, src_hbm, dst_hbm, out_hbm,
              rows0, rows1, rows_t, sbuf0, sbuf1, dbuf0, dbuf1,
              sbuf_t, dbuf_t, acc,
              gsem0, gsem1, ssem0, ssem1, tsem):
    c = lax.axis_index("c")
    s = lax.axis_index("s")
    off = c * N
    rows = (rows0, rows1)
    sbuf = (sbuf0, sbuf1)
    dbuf = (dbuf0, dbuf1)
    gsem = (gsem0, gsem1)
    ssem = (ssem0, ssem1)

    def _zrow(i, _):
        def _z16(j, _):
            rows0[i, pl.ds(j * LN, LN)] = jnp.zeros((LN,), jnp.float32)
            return 0

        lax.fori_loop(0, H // LN, _z16, 0)
        return 0

    lax.fori_loop(0, CH, _zrow, 0)

    def _zero(j, _):
        pltpu.sync_copy(rows0, acc.at[pl.ds(s * ZPT + j * CH, CH)])
        return 0

    lax.fori_loop(0, ZPT // CH, _zero, 0)
    plsc.subcore_barrier()

    def _stage_src(k, p):
        e0 = s * EPT + k * CH
        pltpu.sync_copy(src_hbm.at[pl.ds(e0, CH)], sbuf[p])

        def _add_off(i, _):
            sbuf[p][pl.ds(i * LN, LN)] = sbuf[p][pl.ds(i * LN, LN)] + off
            return 0

        lax.fori_loop(0, CH // LN, _add_off, 0)

    def _stage_dst(k, p):
        e0 = s * EPT + k * CH
        pltpu.sync_copy(dst_hbm.at[pl.ds(e0, CH)], dbuf[p])

    _stage_src(0, 0)
    _stage_dst(0, 0)
    pltpu.async_copy(y_hbm.at[sbuf0], rows0, gsem0)

    def _pipe(kk, _):
        for p in (0, 1):
            q = 1 - p
            k = 2 * kk + p
            pltpu.make_async_copy(y_hbm.at[sbuf[p]], rows[p], gsem[p]).wait()

            @pl.when(k < NCHUNK - 1)
            def _():
                _stage_src(k + 1, q)

            @pl.when(k > 0)
            def _():
                pltpu.make_async_copy(
                    rows[q], acc.at[dbuf[q]], ssem[q]).wait()

            pltpu.async_copy(rows[p], acc.at[dbuf[p]], ssem[p], add=True)

            @pl.when(k < NCHUNK - 1)
            def _():
                _stage_dst(k + 1, q)
                pltpu.async_copy(y_hbm.at[sbuf[q]], rows[q], gsem[q])
        return 0

    lax.fori_loop(0, NCHUNK // 2, _pipe, 0)
    pltpu.make_async_copy(rows1, acc.at[dbuf1], ssem1).wait()

    e0 = s * EPT + NCHUNK * CH
    pltpu.sync_copy(src_hbm.at[pl.ds(e0, TAIL)], sbuf_t)
    pltpu.sync_copy(dst_hbm.at[pl.ds(e0, TAIL)], dbuf_t)

    def _add_off_t(i, _):
        sbuf_t[pl.ds(i * LN, LN)] = sbuf_t[pl.ds(i * LN, LN)] + off
        return 0

    lax.fori_loop(0, TAIL // LN, _add_off_t, 0)
    pltpu.async_copy(y_hbm.at[sbuf_t], rows_t, tsem).wait()
    pltpu.sync_copy(rows_t, acc.at[dbuf_t], add=True)

    plsc.subcore_barrier()
    pltpu.sync_copy(acc.at[pl.ds(s * ZPT, ZPT)],
                    out_hbm.at[pl.ds(c * ACC_ROWS + s * ZPT, ZPT)])


@functools.cache
def _agg_kernel():
    return pl.kernel(
        _agg_body,
        out_type=jax.ShapeDtypeStruct((NC * ACC_ROWS, H), jnp.float32),
        mesh=_mesh(),
        scratch_types=[
            pltpu.VMEM((CH, H), jnp.float32),
            pltpu.VMEM((CH, H), jnp.float32),
            pltpu.VMEM((TAIL, H), jnp.float32),
            pltpu.VMEM((CH,), jnp.int32),
            pltpu.VMEM((CH,), jnp.int32),
            pltpu.VMEM((CH,), jnp.int32),
            pltpu.VMEM((CH,), jnp.int32),
            pltpu.VMEM((TAIL,), jnp.int32),
            pltpu.VMEM((TAIL,), jnp.int32),
            pltpu.VMEM_SHARED((ACC_ROWS, H), jnp.float32),
            pltpu.SemaphoreType.DMA,
            pltpu.SemaphoreType.DMA,
            pltpu.SemaphoreType.DMA,
            pltpu.SemaphoreType.DMA,
            pltpu.SemaphoreType.DMA,
        ],
    )


def _agg_call(y, src2d, dst2d):
    return _agg_kernel()(y, src2d, dst2d)


MB = 1024
NBLK = (N + MB - 1) // MB


def _tc_a_body(x_ref, w_ref, deg_ref, o_ref, dinv_ref):
    deg = deg_ref[...]
    dinv = jnp.where(deg > 0, lax.rsqrt(deg), 0.0)
    y = jnp.dot(x_ref[...], w_ref[...], preferred_element_type=jnp.float32)
    y = y * dinv
    for q in range(NQ):
        o_ref[q] = y[:, q * HQ:(q + 1) * HQ]
    dinv_ref[...] = dinv


def _tc_a(x, w1, deg2):
    return pl.pallas_call(
        _tc_a_body,
        grid=(NBLK,),
        in_specs=[
            pl.BlockSpec((MB, D), lambda i: (i, 0)),
            pl.BlockSpec((D, D), lambda i: (0, 0)),
            pl.BlockSpec((MB, 1), lambda i: (i, 0)),
        ],
        out_specs=[
            pl.BlockSpec((NQ, MB, HQ), lambda i: (0, i, 0)),
            pl.BlockSpec((MB, 1), lambda i: (i, 0)),
        ],
        out_shape=[
            jax.ShapeDtypeStruct((NQ, N, HQ), jnp.float32),
            jax.ShapeDtypeStruct((N, 1), jnp.float32),
        ],
    )(x, w1, deg2)


def _tc_b_body(agg_ref, dinv_ref, b1_ref, a1_ref, w2_ref, rw_ref, rb_ref,
               y2_ref, r_ref):
    dinv = dinv_ref[...]
    pre = jnp.concatenate([agg_ref[q] for q in range(NQ)], axis=1) * dinv \
        + b1_ref[...]
    h = jnp.where(pre >= 0, pre, a1_ref[...] * pre)
    y2 = jnp.dot(h, w2_ref[...], preferred_element_type=jnp.float32) * dinv
    for q in range(NQ):
        y2_ref[q] = y2[:, q * HQ:(q + 1) * HQ]
    r_ref[...] = jnp.dot(h, rw_ref[...], preferred_element_type=jnp.float32) \
        + rb_ref[...]


def _tc_b(agg1, dinv2, b1, a1, w2, rw, rb):
    return pl.pallas_call(
        _tc_b_body,
        grid=(NBLK,),
        in_specs=[
            pl.BlockSpec((NQ, MB, HQ), lambda i: (0, i, 0)),
            pl.BlockSpec((MB, 1), lambda i: (i, 0)),
            pl.BlockSpec((1, D), lambda i: (0, 0)),
            pl.BlockSpec((1, D), lambda i: (0, 0)),
            pl.BlockSpec((D, D), lambda i: (0, 0)),
            pl.BlockSpec((D, D), lambda i: (0, 0)),
            pl.BlockSpec((1, D), lambda i: (0, 0)),
        ],
        out_specs=[
            pl.BlockSpec((NQ, MB, HQ), lambda i: (0, i, 0)),
            pl.BlockSpec((MB, D), lambda i: (i, 0)),
        ],
        out_shape=[
            jax.ShapeDtypeStruct((NQ, N, HQ), jnp.float32),
            jax.ShapeDtypeStruct((N, D), jnp.float32),
        ],
    )(agg1, dinv2, b1, a1, w2, rw, rb)


def _tc_c_body(agg_ref, dinv_ref, b2_ref, a2_ref, r_ref, wc_ref, bc_ref, o_ref):
    dinv = dinv_ref[...]
    pre = jnp.concatenate([agg_ref[q] for q in range(NQ)], axis=1) * dinv \
        + b2_ref[...] + r_ref[...]
    h2 = jnp.where(pre >= 0, pre, a2_ref[...] * pre)
    o_ref[...] = jnp.dot(h2, wc_ref[...], preferred_element_type=jnp.float32) \
        + bc_ref[...]


def _tc_c(agg2, dinv2, b2, a2, r, wc, bc):
    return pl.pallas_call(
        _tc_c_body,
        grid=(NBLK,),
        in_specs=[
            pl.BlockSpec((NQ, MB, HQ), lambda i: (0, i, 0)),
            pl.BlockSpec((MB, 1), lambda i: (i, 0)),
            pl.BlockSpec((1, D), lambda i: (0, 0)),
            pl.BlockSpec((1, D), lambda i: (0, 0)),
            pl.BlockSpec((MB, D), lambda i: (i, 0)),
            pl.BlockSpec((D, C), lambda i: (0, 0)),
            pl.BlockSpec((1, C), lambda i: (0, 0)),
        ],
        out_specs=pl.BlockSpec((MB, C), lambda i: (i, 0)),
        out_shape=jax.ShapeDtypeStruct((N, C), jnp.float32),
    )(agg2, dinv2, b2, a2, r, wc, bc)


def kernel(x, edge_index, W1, b1, a1, W2, b2, a2, resW, resb, Wc, bc):
    src = edge_index[0]
    dst = edge_index[1]
    deg = _deg_call(dst)
    deg2 = deg[:N].reshape(N, 1)
    y1, dinv2 = _tc_a(x, W1, deg2)
    agg1 = _agg_call(y1.reshape(NQ * N, HQ), src,
                     dst).reshape(NQ, ACC_ROWS, HQ)
    y2, r = _tc_b(agg1, dinv2, b1.reshape(1, D), a1.reshape(1, D),
                  W2, resW, resb.reshape(1, D))
    agg2 = _agg_call(y2.reshape(NQ * N, HQ), src,
                     dst).reshape(NQ, ACC_ROWS, HQ)
    return _tc_c(agg2, dinv2, b2.reshape(1, D), a2.reshape(1, D),
                 r, Wc, bc.reshape(1, C))
